# baseline (speedup 1.0000x reference)
import functools

import jax
import jax.numpy as jnp
import numpy as np
from jax import lax
from jax.experimental import pallas as pl
from jax.experimental.pallas import tpu as pltpu
from jax.experimental.pallas import tpu_sc as plsc

B = 16384
H = 200
D = 128
HID = 32
NCLS = 100

NC, NS = 2, 16
NW = NC * NS
BPW = B // NW
SB = 64
NBLK = BPW // SB


NBUF = 4


def _sc_pool_body(ids_hbm, table_hbm, out_hbm, idx_v, rows_v, out_v,
                  sem0, sem1, sem2, sem3):
    wid = lax.axis_index("s") * NC + lax.axis_index("c")
    base = wid * BPW
    sems = (sem0, sem1, sem2, sem3)

    def blk_body(blk, carry):
        row0 = base + blk * SB
        pltpu.sync_copy(ids_hbm.at[pl.ds(row0, SB)], idx_v)

        def gather_descs(s, p):
            return (
                pltpu.make_async_copy(
                    table_hbm.at[idx_v.at[s, pl.ds(0, 128)]],
                    rows_v.at[p, pl.ds(0, 128)], sems[p]),
                pltpu.make_async_copy(
                    table_hbm.at[idx_v.at[s, pl.ds(128, 72)]],
                    rows_v.at[p, pl.ds(128, 72)], sems[p]),
            )

        def start(s, p):
            for cp in gather_descs(s, p):
                cp.start()

        def wait(s, p):
            for cp in gather_descs(s, p):
                cp.wait()

        def accum(s, p):
            def red(r, acc):
                return tuple(acc[j] + rows_v[p, r, pl.ds(16 * j, 16)]
                             for j in range(D // 16))

            acc = lax.fori_loop(
                0, H, red,
                tuple(jnp.zeros((16,), jnp.float32) for _ in range(D // 16)),
                unroll=4)
            for j in range(D // 16):
                out_v[s, pl.ds(16 * j, 16)] = acc[j]

        for k in range(NBUF):
            start(k, k)

        def ring_body(t, carry):
            s0 = t * NBUF
            for k in range(NBUF):
                wait(s0 + k, k)
                accum(s0 + k, k)

                @pl.when(s0 + k + NBUF < SB)
                def _():
                    start(s0 + k + NBUF, k)
            return carry

        lax.fori_loop(0, SB // NBUF, ring_body, 0)
        pltpu.sync_copy(out_v, out_hbm.at[pl.ds(row0, SB)])
        return carry

    lax.fori_loop(0, NBLK, blk_body, 0)


_sc_pool = functools.partial(
    pl.kernel,
    out_type=jax.ShapeDtypeStruct((B, D), jnp.float32),
    mesh=plsc.VectorSubcoreMesh(core_axis_name="c", subcore_axis_name="s"),
    scratch_types=[
        pltpu.VMEM((SB, H), jnp.int32),
        pltpu.VMEM((NBUF, H, D), jnp.float32),
        pltpu.VMEM((SB, D), jnp.float32),
        pltpu.SemaphoreType.DMA,
        pltpu.SemaphoreType.DMA,
        pltpu.SemaphoreType.DMA,
        pltpu.SemaphoreType.DMA,
    ],
)(_sc_pool_body)


def _mlp_body(sums_ref, w1_ref, b1_ref, w2_ref, b2_ref, out_ref):
    pooled = sums_ref[...] * jnp.float32(1.0 / H)
    h = jnp.dot(pooled, w1_ref[...], preferred_element_type=jnp.float32)
    h = jnp.maximum(h + b1_ref[...], 0.0)
    out = jnp.dot(h, w2_ref[...], preferred_element_type=jnp.float32)
    out_ref[...] = out + b2_ref[...]


def _mlp(sums, W1, b1, W2, b2):
    MB = 2048
    return pl.pallas_call(
        _mlp_body,
        out_shape=jax.ShapeDtypeStruct((B, NCLS), jnp.float32),
        grid=(B // MB,),
        in_specs=[
            pl.BlockSpec((MB, D), lambda i: (i, 0)),
            pl.BlockSpec((D, HID), lambda i: (0, 0)),
            pl.BlockSpec((1, HID), lambda i: (0, 0)),
            pl.BlockSpec((HID, NCLS), lambda i: (0, 0)),
            pl.BlockSpec((1, NCLS), lambda i: (0, 0)),
        ],
        out_specs=pl.BlockSpec((MB, NCLS), lambda i: (i, 0)),
    )(sums, W1.T, b1.reshape(1, HID), W2.T, b2.reshape(1, NCLS))


def kernel(input_ids, emb_table, W1, b1, W2, b2):
    ids = input_ids.astype(jnp.int32)
    sums = _sc_pool(ids, emb_table)
    return _mlp(sums, W1, b1, W2, b2)


# speedup vs baseline: 1.0496x; 1.0496x over previous
import functools

import jax
import jax.numpy as jnp
from jax import lax
from jax.experimental import pallas as pl
from jax.experimental.pallas import tpu as pltpu
from jax.experimental.pallas import tpu_sc as plsc

B = 16384
H = 200
D = 128
HID = 32
NCLS = 100

NC, NS = 2, 16
NW = NC * NS
BPW = B // NW
SB = 32
NBLK = BPW // SB
NBUF = 4


def _sc_pool_body(ids_hbm, table_hbm, out_hbm, idx_v, rows_v, out_v,
                  sem0, sem1, sem2, sem3, sem_idx, sem_o0, sem_o1):
    wid = lax.axis_index("s") * NC + lax.axis_index("c")
    base = wid * BPW
    sems = (sem0, sem1, sem2, sem3)
    osems = (sem_o0, sem_o1)

    def idx_desc(blk, pb):
        return pltpu.make_async_copy(
            ids_hbm.at[pl.ds(base + blk * SB, SB)], idx_v.at[pb], sem_idx)

    def out_desc(blk, pb):
        return pltpu.make_async_copy(
            out_v.at[pb], out_hbm.at[pl.ds(base + blk * SB, SB)], osems[pb])

    def gather_descs(pb, s, k):
        return (
            pltpu.make_async_copy(
                table_hbm.at[idx_v.at[pb, s, pl.ds(0, 128)]],
                rows_v.at[k, pl.ds(0, 128)], sems[k]),
            pltpu.make_async_copy(
                table_hbm.at[idx_v.at[pb, s, pl.ds(128, 72)]],
                rows_v.at[k, pl.ds(128, 72)], sems[k]),
        )

    def start(pb, s, k):
        for cp in gather_descs(pb, s, k):
            cp.start()

    def wait(pb, s, k):
        for cp in gather_descs(pb, s, k):
            cp.wait()

    def accum(pb, s, k):
        def red(r, acc):
            return tuple(acc[j] + rows_v[k, r, pl.ds(16 * j, 16)]
                         for j in range(D // 16))

        acc = lax.fori_loop(
            0, H, red,
            tuple(jnp.zeros((16,), jnp.float32) for _ in range(D // 16)),
            unroll=4)
        for j in range(D // 16):
            out_v[pb, s, pl.ds(16 * j, 16)] = acc[j]

    idx_desc(0, 0).start()
    idx_desc(0, 0).wait()
    for k in range(NBUF):
        start(0, k, k)

    def one_block(blk, pb):
        npb = 1 - pb

        @pl.when(blk + 1 < NBLK)
        def _():
            idx_desc(blk + 1, npb).start()

        @pl.when(blk >= 2)
        def _():
            out_desc(blk - 2, pb).wait()

        def phase_a(g, carry):
            s0 = g * NBUF
            for k in range(NBUF):
                wait(pb, s0 + k, k)
                accum(pb, s0 + k, k)
                start(pb, s0 + k + NBUF, k)
            return carry

        lax.fori_loop(0, SB // NBUF - 1, phase_a, 0)

        @pl.when(blk + 1 < NBLK)
        def _():
            idx_desc(blk + 1, npb).wait()

        for k in range(NBUF):
            s = SB - NBUF + k
            wait(pb, s, k)
            accum(pb, s, k)

            @pl.when(blk + 1 < NBLK)
            def _():
                start(npb, k, k)

        out_desc(blk, pb).start()

    def blk_pair_body(i, carry):
        one_block(2 * i, 0)
        one_block(2 * i + 1, 1)
        return carry

    lax.fori_loop(0, NBLK // 2, blk_pair_body, 0)
    out_desc(NBLK - 2, (NBLK - 2) % 2).wait()
    out_desc(NBLK - 1, (NBLK - 1) % 2).wait()


_sc_pool = functools.partial(
    pl.kernel,
    out_type=jax.ShapeDtypeStruct((B, D), jnp.float32),
    mesh=plsc.VectorSubcoreMesh(core_axis_name="c", subcore_axis_name="s"),
    scratch_types=[
        pltpu.VMEM((2, SB, H), jnp.int32),
        pltpu.VMEM((NBUF, H, D), jnp.float32),
        pltpu.VMEM((2, SB, D), jnp.float32),
        pltpu.SemaphoreType.DMA,
        pltpu.SemaphoreType.DMA,
        pltpu.SemaphoreType.DMA,
        pltpu.SemaphoreType.DMA,
        pltpu.SemaphoreType.DMA,
        pltpu.SemaphoreType.DMA,
        pltpu.SemaphoreType.DMA,
    ],
)(_sc_pool_body)


def _mlp_body(sums_ref, w1_ref, b1_ref, w2_ref, b2_ref, out_ref):
    pooled = sums_ref[...] * jnp.float32(1.0 / H)
    h = jnp.dot(pooled, w1_ref[...], preferred_element_type=jnp.float32)
    h = jnp.maximum(h + b1_ref[...], 0.0)
    out = jnp.dot(h, w2_ref[...], preferred_element_type=jnp.float32)
    out_ref[...] = out + b2_ref[...]


def _mlp(sums, W1, b1, W2, b2):
    MB = 2048
    return pl.pallas_call(
        _mlp_body,
        out_shape=jax.ShapeDtypeStruct((B, NCLS), jnp.float32),
        grid=(B // MB,),
        in_specs=[
            pl.BlockSpec((MB, D), lambda i: (i, 0)),
            pl.BlockSpec((D, HID), lambda i: (0, 0)),
            pl.BlockSpec((1, HID), lambda i: (0, 0)),
            pl.BlockSpec((HID, NCLS), lambda i: (0, 0)),
            pl.BlockSpec((1, NCLS), lambda i: (0, 0)),
        ],
        out_specs=pl.BlockSpec((MB, NCLS), lambda i: (i, 0)),
    )(sums, W1.T, b1.reshape(1, HID), W2.T, b2.reshape(1, NCLS))


def kernel(input_ids, emb_table, W1, b1, W2, b2):
    ids = input_ids.astype(jnp.int32)
    sums = _sc_pool(ids, emb_table)
    return _mlp(sums, W1, b1, W2, b2)


# speedup vs baseline: 1.0522x; 1.0025x over previous
import functools

import jax
import jax.numpy as jnp
from jax import lax
from jax.experimental import pallas as pl
from jax.experimental.pallas import tpu as pltpu
from jax.experimental.pallas import tpu_sc as plsc

B = 16384
H = 200
D = 128
HID = 32
NCLS = 100

NC, NS = 2, 16
NW = NC * NS
BPW = B // NW
SB = 32
NBLK = BPW // SB
NBUF = 4


def _sc_pool_body(ids_hbm, table_hbm, out_hbm, idx_v, rows_v, out_v,
                  sem0, sem1, sem2, sem3, sem_idx, sem_o0, sem_o1):
    wid = lax.axis_index("s") * NC + lax.axis_index("c")
    base = wid * BPW
    sems = (sem0, sem1, sem2, sem3)
    osems = (sem_o0, sem_o1)

    def idx_desc(blk, pb):
        return pltpu.make_async_copy(
            ids_hbm.at[pl.ds(base + blk * SB, SB)], idx_v.at[pb], sem_idx)

    def out_desc(blk, pb):
        return pltpu.make_async_copy(
            out_v.at[pb], out_hbm.at[pl.ds(base + blk * SB, SB)], osems[pb])

    _SPLITS = ((0, 64), (64, 64), (128, 72))

    def gather_descs(pb, s, k):
        return tuple(
            pltpu.make_async_copy(
                table_hbm.at[idx_v.at[pb, s, pl.ds(o, n)]],
                rows_v.at[k, pl.ds(o, n)], sems[k])
            for o, n in _SPLITS)

    def start(pb, s, k):
        for cp in gather_descs(pb, s, k):
            cp.start()

    def wait(pb, s, k):
        for cp in gather_descs(pb, s, k):
            cp.wait()

    def accum(pb, s, k):
        def red(r, acc):
            return tuple(acc[j] + rows_v[k, r, pl.ds(16 * j, 16)]
                         for j in range(D // 16))

        acc = lax.fori_loop(
            0, H, red,
            tuple(jnp.zeros((16,), jnp.float32) for _ in range(D // 16)),
            unroll=4)
        for j in range(D // 16):
            out_v[pb, s, pl.ds(16 * j, 16)] = acc[j]

    idx_desc(0, 0).start()
    idx_desc(0, 0).wait()
    for k in range(NBUF):
        start(0, k, k)

    def one_block(blk, pb):
        npb = 1 - pb

        @pl.when(blk + 1 < NBLK)
        def _():
            idx_desc(blk + 1, npb).start()

        @pl.when(blk >= 2)
        def _():
            out_desc(blk - 2, pb).wait()

        def phase_a(g, carry):
            s0 = g * NBUF
            for k in range(NBUF):
                wait(pb, s0 + k, k)
                accum(pb, s0 + k, k)
                start(pb, s0 + k + NBUF, k)
            return carry

        lax.fori_loop(0, SB // NBUF - 1, phase_a, 0)

        @pl.when(blk + 1 < NBLK)
        def _():
            idx_desc(blk + 1, npb).wait()

        for k in range(NBUF):
            s = SB - NBUF + k
            wait(pb, s, k)
            accum(pb, s, k)

            @pl.when(blk + 1 < NBLK)
            def _():
                start(npb, k, k)

        out_desc(blk, pb).start()

    def blk_pair_body(i, carry):
        one_block(2 * i, 0)
        one_block(2 * i + 1, 1)
        return carry

    lax.fori_loop(0, NBLK // 2, blk_pair_body, 0)
    out_desc(NBLK - 2, (NBLK - 2) % 2).wait()
    out_desc(NBLK - 1, (NBLK - 1) % 2).wait()


_sc_pool = functools.partial(
    pl.kernel,
    out_type=jax.ShapeDtypeStruct((B, D), jnp.float32),
    mesh=plsc.VectorSubcoreMesh(core_axis_name="c", subcore_axis_name="s"),
    scratch_types=[
        pltpu.VMEM((2, SB, H), jnp.int32),
        pltpu.VMEM((NBUF, H, D), jnp.float32),
        pltpu.VMEM((2, SB, D), jnp.float32),
        pltpu.SemaphoreType.DMA,
        pltpu.SemaphoreType.DMA,
        pltpu.SemaphoreType.DMA,
        pltpu.SemaphoreType.DMA,
        pltpu.SemaphoreType.DMA,
        pltpu.SemaphoreType.DMA,
        pltpu.SemaphoreType.DMA,
    ],
)(_sc_pool_body)


def _mlp_body(sums_ref, w1_ref, b1_ref, w2_ref, b2_ref, out_ref):
    pooled = sums_ref[...] * jnp.float32(1.0 / H)
    h = jnp.dot(pooled, w1_ref[...], preferred_element_type=jnp.float32)
    h = jnp.maximum(h + b1_ref[...], 0.0)
    out = jnp.dot(h, w2_ref[...], preferred_element_type=jnp.float32)
    out_ref[...] = out + b2_ref[...]


def _mlp(sums, W1, b1, W2, b2):
    MB = 2048
    return pl.pallas_call(
        _mlp_body,
        out_shape=jax.ShapeDtypeStruct((B, NCLS), jnp.float32),
        grid=(B // MB,),
        in_specs=[
            pl.BlockSpec((MB, D), lambda i: (i, 0)),
            pl.BlockSpec((D, HID), lambda i: (0, 0)),
            pl.BlockSpec((1, HID), lambda i: (0, 0)),
            pl.BlockSpec((HID, NCLS), lambda i: (0, 0)),
            pl.BlockSpec((1, NCLS), lambda i: (0, 0)),
        ],
        out_specs=pl.BlockSpec((MB, NCLS), lambda i: (i, 0)),
    )(sums, W1.T, b1.reshape(1, HID), W2.T, b2.reshape(1, NCLS))


def kernel(input_ids, emb_table, W1, b1, W2, b2):
    ids = input_ids.astype(jnp.int32)
    sums = _sc_pool(ids, emb_table)
    return _mlp(sums, W1, b1, W2, b2)
